# Initial kernel scaffold; baseline (speedup 1.0000x reference)
import jax
import jax.numpy as jnp
from jax.experimental import pallas as pl


def kernel(x_t0, x_t1, x_t2, edge_attr_t0, edge_attr_t1, edge_attr_t2, initial_weights, W_ih, W_hh, b_ih, b_hh, W1, b1, W2, b2, edge_index_t0, edge_index_t1, edge_index_t2):
    raise NotImplementedError("write your pallas kernel here")


# speedup vs baseline: 6.5488x; 6.5488x over previous
import functools

import jax
import jax.numpy as jnp
from jax import lax
from jax.experimental import pallas as pl
from jax.experimental.pallas import tpu as pltpu
from jax.experimental.pallas import tpu_sc as plsc

N = 50000
E = 800000
D_IN = 32
D_HID = 64
D_EDGE = 4
FLAT = D_IN * D_HID
G4 = 4 * FLAT

N_PAD = 50048
E_PAD = 802816
NPT = N_PAD // 16
EPT32 = E_PAD // 32
EPT16 = E_PAD // 16
CK = 128
DEG_CK = 1568

_mesh = plsc.VectorSubcoreMesh(core_axis_name="c", subcore_axis_name="s")
_sc_params = pltpu.CompilerParams(needs_layout_passes=False,
                                  use_tc_tiling_on_sc=False)


_LB = 512
_NB = G4 // _LB


def _lstm_body(w0_ref, wih_ref, whh_ref, bih_ref, bhh_ref, out_ref,
               gates_ref, h_ref, c_ref):
    s = pl.program_id(0)
    b = pl.program_id(1)

    @pl.when((s == 0) & (b == 0))
    def _():
        h_ref[...] = w0_ref[...]
        c_ref[...] = jnp.zeros_like(c_ref)

    scale = jnp.where(s == 0, 0.0, 1.0).astype(jnp.float32)
    wblk = wih_ref[...] + scale * whh_ref[...]
    xin = h_ref[...]
    g = lax.dot_general(xin, wblk, (((1,), (1,)), ((), ())),
                        preferred_element_type=jnp.float32)
    gates_ref[pl.ds(b, 1), :] = g + bih_ref[0] + bhh_ref[0]

    @pl.when(b == _NB - 1)
    def _():
        gs = gates_ref[...].reshape(4, FLAT)
        i_ = jax.nn.sigmoid(gs[0:1])
        f_ = jax.nn.sigmoid(gs[1:2])
        g_ = jnp.tanh(gs[2:3])
        o_ = jax.nn.sigmoid(gs[3:4])
        c2 = f_ * c_ref[...] + i_ * g_
        h2 = o_ * jnp.tanh(c2)
        c_ref[...] = c2
        h_ref[...] = h2

        @pl.when(s == 2)
        def _():
            out_ref[...] = h2


def _lstm(w0, W_ih, W_hh, b_ih, b_hh):
    bih2 = b_ih.reshape(_NB, 1, _LB)
    bhh2 = b_hh.reshape(_NB, 1, _LB)
    h = pl.pallas_call(
        _lstm_body,
        grid=(3, _NB),
        in_specs=[
            pl.BlockSpec((1, FLAT), lambda s, b: (0, 0)),
            pl.BlockSpec((_LB, FLAT), lambda s, b: (b, 0)),
            pl.BlockSpec((_LB, FLAT), lambda s, b: (jnp.where(s == 0, 0, b), 0)),
            pl.BlockSpec((1, 1, _LB), lambda s, b: (b, 0, 0)),
            pl.BlockSpec((1, 1, _LB), lambda s, b: (b, 0, 0)),
        ],
        out_specs=pl.BlockSpec((1, FLAT), lambda s, b: (0, 0)),
        out_shape=jax.ShapeDtypeStruct((1, FLAT), jnp.float32),
        scratch_shapes=[
            pltpu.VMEM((_NB, _LB), jnp.float32),
            pltpu.VMEM((1, FLAT), jnp.float32),
            pltpu.VMEM((1, FLAT), jnp.float32),
        ],
    )(w0.reshape(1, FLAT), W_ih, W_hh, bih2, bhh2)
    return h.reshape(D_IN, D_HID)


@functools.partial(
    pl.kernel,
    out_type=jax.ShapeDtypeStruct((32, N_PAD), jnp.float32),
    mesh=_mesh,
    compiler_params=_sc_params,
    scratch_types=[
        pltpu.VMEM((N_PAD,), jnp.float32),
        pltpu.VMEM((DEG_CK,), jnp.int32),
    ],
)
def _deg_kernel(col_hbm, out_hbm, deg_v, idx_v):
    c = lax.axis_index("c")
    s = lax.axis_index("s")
    wid = s * 2 + c
    zeros16 = jnp.zeros((16,), jnp.float32)
    ones16 = jnp.ones((16,), jnp.float32)

    def zb(i, carry):
        deg_v[pl.ds(i * 16, 16)] = zeros16
        return carry

    lax.fori_loop(0, N_PAD // 16, zb, 0)

    base = wid * EPT32

    def chunk(k, carry):
        pltpu.sync_copy(col_hbm.at[pl.ds(base + k * DEG_CK, DEG_CK)], idx_v)

        def inner(j, cc):
            idx16 = idx_v[pl.ds(j * 16, 16)]
            plsc.addupdate_scatter(deg_v, [idx16], ones16)
            return cc

        lax.fori_loop(0, DEG_CK // 16, inner, 0)
        return carry

    lax.fori_loop(0, EPT32 // DEG_CK, chunk, 0)
    pltpu.sync_copy(deg_v, out_hbm.at[wid])


_NBLK = 2944


def _prep_body(x_ref, degp_ref, w_ref, ya_ref, yb_ref, dis_ref):
    deg = jnp.sum(degp_ref[...], axis=0) + 1.0
    dis = lax.rsqrt(deg)
    xw = jnp.dot(x_ref[...], w_ref[...], preferred_element_type=jnp.float32)
    y = dis[:, None] * xw
    ya_ref[...] = y[:, :32]
    yb_ref[...] = y[:, 32:]
    dis_ref[...] = dis[:, None]


def _prep(x_pad, deg_partials, W):
    return pl.pallas_call(
        _prep_body,
        grid=(N_PAD // _NBLK,),
        in_specs=[
            pl.BlockSpec((_NBLK, D_IN), lambda i: (i, 0)),
            pl.BlockSpec((32, _NBLK), lambda i: (0, i)),
            pl.BlockSpec((D_IN, D_HID), lambda i: (0, 0)),
        ],
        out_specs=[
            pl.BlockSpec((_NBLK, 32), lambda i: (i, 0)),
            pl.BlockSpec((_NBLK, 32), lambda i: (i, 0)),
            pl.BlockSpec((_NBLK, 1), lambda i: (i, 0)),
        ],
        out_shape=[
            jax.ShapeDtypeStruct((N_PAD, 32), jnp.float32),
            jax.ShapeDtypeStruct((N_PAD, 32), jnp.float32),
            jax.ShapeDtypeStruct((N_PAD, 1), jnp.float32),
        ],
    )(x_pad, deg_partials, W)


_ZB = 782


@functools.partial(
    pl.kernel,
    out_type=(
        jax.ShapeDtypeStruct((N_PAD, 32), jnp.float32),
        jax.ShapeDtypeStruct((N_PAD, 32), jnp.float32),
    ),
    mesh=_mesh,
    compiler_params=_sc_params,
    scratch_types=[
        pltpu.VMEM((_ZB, 32), jnp.float32),
        pltpu.VMEM((CK,), jnp.int32),
        pltpu.VMEM((CK,), jnp.int32),
        pltpu.VMEM((CK, 32), jnp.float32),
        pltpu.VMEM_SHARED((N_PAD, 32), jnp.float32),
        pltpu.SemaphoreType.DMA,
    ],
)
def _agg_kernel(ya_hbm, yb_hbm, row_hbm, col_hbm, outa_hbm, outb_hbm,
                zbuf, ridx, cidx, gbuf, accum, sem):
    c = lax.axis_index("c")
    s = lax.axis_index("s")
    zeros16 = jnp.zeros((16,), jnp.float32)

    def zrow(i, carry):
        zbuf[i, pl.ds(0, 16)] = zeros16
        zbuf[i, pl.ds(16, 16)] = zeros16
        return carry

    lax.fori_loop(0, _ZB, zrow, 0)

    def zslice(q, carry):
        pltpu.sync_copy(zbuf, accum.at[pl.ds(s * NPT + q * _ZB, _ZB)])
        return carry

    lax.fori_loop(0, NPT // _ZB, zslice, 0)
    plsc.subcore_barrier()

    base = s * EPT16

    def chunk(k, carry):
        off = base + k * CK
        pltpu.sync_copy(row_hbm.at[pl.ds(off, CK)], ridx)
        pltpu.sync_copy(col_hbm.at[pl.ds(off, CK)], cidx)

        @pl.when(c == 0)
        def _():
            pltpu.async_copy(ya_hbm.at[ridx], gbuf, sem).wait()

        @pl.when(c == 1)
        def _():
            pltpu.async_copy(yb_hbm.at[ridx], gbuf, sem).wait()

        pltpu.sync_copy(gbuf, accum.at[cidx], add=True)
        return carry

    lax.fori_loop(0, EPT16 // CK, chunk, 0)
    plsc.subcore_barrier()

    @pl.when(c == 0)
    def _():
        pltpu.sync_copy(accum.at[pl.ds(s * NPT, NPT)],
                        outa_hbm.at[pl.ds(s * NPT, NPT)])

    @pl.when(c == 1)
    def _():
        pltpu.sync_copy(accum.at[pl.ds(s * NPT, NPT)],
                        outb_hbm.at[pl.ds(s * NPT, NPT)])


def _node_mlp_body(agga_ref, aggb_ref, ya_ref, yb_ref, dis_ref,
                   w1s_ref, w1d_ref, b1_ref, asrc_ref, adst_ref):
    agg = jnp.concatenate(
        [agga_ref[...] + ya_ref[...], aggb_ref[...] + yb_ref[...]], axis=1)
    emb = jnp.maximum(dis_ref[...] * agg, 0.0)
    asrc_ref[...] = jnp.dot(emb, w1s_ref[...],
                            preferred_element_type=jnp.float32)
    adst_ref[...] = jnp.dot(emb, w1d_ref[...],
                            preferred_element_type=jnp.float32) + b1_ref[...]


def _node_mlp(agg_a, agg_b, y_a, y_b, dis, W1s, W1d, b1):
    nspec = pl.BlockSpec((_NBLK, 32), lambda i: (i, 0))
    return pl.pallas_call(
        _node_mlp_body,
        grid=(N_PAD // _NBLK,),
        in_specs=[
            nspec, nspec, nspec, nspec,
            pl.BlockSpec((_NBLK, 1), lambda i: (i, 0)),
            pl.BlockSpec((D_HID, D_HID), lambda i: (0, 0)),
            pl.BlockSpec((D_HID, D_HID), lambda i: (0, 0)),
            pl.BlockSpec((1, D_HID), lambda i: (0, 0)),
        ],
        out_specs=[
            pl.BlockSpec((_NBLK, D_HID), lambda i: (i, 0)),
            pl.BlockSpec((_NBLK, D_HID), lambda i: (i, 0)),
        ],
        out_shape=[
            jax.ShapeDtypeStruct((N_PAD, D_HID), jnp.float32),
            jax.ShapeDtypeStruct((N_PAD, D_HID), jnp.float32),
        ],
    )(agg_a, agg_b, y_a, y_b, dis, W1s, W1d, b1.reshape(1, D_HID))


@functools.partial(
    pl.kernel,
    out_type=jax.ShapeDtypeStruct((E_PAD, D_HID), jnp.float32),
    mesh=_mesh,
    compiler_params=_sc_params,
    scratch_types=[
        pltpu.VMEM((CK,), jnp.int32),
        pltpu.VMEM((CK,), jnp.int32),
        pltpu.VMEM((CK, D_HID), jnp.float32),
        pltpu.VMEM((CK, D_HID), jnp.float32),
        pltpu.SemaphoreType.DMA,
        pltpu.SemaphoreType.DMA,
    ],
)
def _edge_gather_kernel(asrc_hbm, adst_hbm, src_hbm, dst_hbm, out_hbm,
                        sidx, didx, buf1, buf2, sem1, sem2):
    c = lax.axis_index("c")
    s = lax.axis_index("s")
    wid = s * 2 + c
    base = wid * EPT32

    def chunk(k, carry):
        off = base + k * CK
        pltpu.sync_copy(src_hbm.at[pl.ds(off, CK)], sidx)
        pltpu.sync_copy(dst_hbm.at[pl.ds(off, CK)], didx)
        cp1 = pltpu.async_copy(asrc_hbm.at[sidx], buf1, sem1)
        cp2 = pltpu.async_copy(adst_hbm.at[didx], buf2, sem2)
        cp1.wait()
        cp2.wait()

        def add_row(j, cc):
            for l in range(D_HID // 16):
                sl = pl.ds(l * 16, 16)
                buf1[j, sl] = buf1[j, sl] + buf2[j, sl]
            return cc

        lax.fori_loop(0, CK, add_row, 0)
        pltpu.sync_copy(buf1, out_hbm.at[pl.ds(off, CK)])
        return carry

    lax.fori_loop(0, EPT32 // CK, chunk, 0)


_EBLK = 6272


def _edge_mlp_body(g_ref, ea_ref, w1e_ref, w2_ref, b2_ref, out_ref):
    eproj = jnp.dot(ea_ref[...], w1e_ref[...],
                    preferred_element_type=jnp.float32)
    hid = jnp.maximum(g_ref[...] + eproj, 0.0)
    out_ref[...] = jnp.dot(hid, w2_ref[...],
                           preferred_element_type=jnp.float32) + b2_ref[...]


def _edge_mlp(g, eattr_pad, W1e, W2, b2):
    return pl.pallas_call(
        _edge_mlp_body,
        grid=(E_PAD // _EBLK,),
        in_specs=[
            pl.BlockSpec((_EBLK, D_HID), lambda i: (i, 0)),
            pl.BlockSpec((_EBLK, D_EDGE), lambda i: (i, 0)),
            pl.BlockSpec((D_EDGE, D_HID), lambda i: (0, 0)),
            pl.BlockSpec((D_HID, 1), lambda i: (0, 0)),
            pl.BlockSpec((1, 1), lambda i: (0, 0)),
        ],
        out_specs=pl.BlockSpec((_EBLK, 1), lambda i: (i, 0)),
        out_shape=jax.ShapeDtypeStruct((E_PAD, 1), jnp.float32),
    )(g, eattr_pad, W1e, W2, b2.reshape(1, 1))


def kernel(x_t0, x_t1, x_t2, edge_attr_t0, edge_attr_t1, edge_attr_t2,
           initial_weights, W_ih, W_hh, b_ih, b_hh, W1, b1, W2, b2,
           edge_index_t0, edge_index_t1, edge_index_t2):
    row = edge_index_t2[0]
    col = edge_index_t2[1]
    pad_e = E_PAD - E
    row_pad = jnp.concatenate([row, jnp.zeros((pad_e,), jnp.int32)])
    col_pad = jnp.concatenate([col, jnp.full((pad_e,), N, jnp.int32)])
    x_pad = jnp.pad(x_t2, ((0, N_PAD - N), (0, 0)))
    eattr_pad = jnp.pad(edge_attr_t2, ((0, pad_e), (0, 0)))
    W1s = W1[:D_HID]
    W1d = W1[D_HID:2 * D_HID]
    W1e = W1[2 * D_HID:]

    W = _lstm(initial_weights, W_ih, W_hh, b_ih, b_hh)
    deg_partials = _deg_kernel(col_pad)
    y_a, y_b, dis = _prep(x_pad, deg_partials, W)
    agg_a, agg_b = _agg_kernel(y_a, y_b, row_pad, col_pad)
    a_src, a_dst = _node_mlp(agg_a, agg_b, y_a, y_b, dis, W1s, W1d, b1)
    g = _edge_gather_kernel(a_src, a_dst, row_pad, col_pad)
    logits = _edge_mlp(g, eattr_pad, W1e, W2, b2)
    return logits[:E, 0]


# speedup vs baseline: 8.4290x; 1.2871x over previous
import functools

import jax
import jax.numpy as jnp
from jax import lax
from jax.experimental import pallas as pl
from jax.experimental.pallas import tpu as pltpu
from jax.experimental.pallas import tpu_sc as plsc

N = 50000
E = 800000
D_IN = 32
D_HID = 64
D_EDGE = 4
FLAT = D_IN * D_HID
G4 = 4 * FLAT

N_PAD = 50048
E_PAD = 802816
NPT = N_PAD // 16
EPT32 = E_PAD // 32
EPT16 = E_PAD // 16
CK = 128
DEG_CK = 1568

_mesh = plsc.VectorSubcoreMesh(core_axis_name="c", subcore_axis_name="s")
_sc_params = pltpu.CompilerParams(needs_layout_passes=False,
                                  use_tc_tiling_on_sc=False)


_LB = 512
_NB = G4 // _LB


def _lstm_body(w0_ref, wih_ref, whh_ref, bih_ref, bhh_ref, out_ref,
               gates_ref, h_ref, c_ref):
    s = pl.program_id(0)
    b = pl.program_id(1)

    @pl.when((s == 0) & (b == 0))
    def _():
        h_ref[...] = w0_ref[...]
        c_ref[...] = jnp.zeros_like(c_ref)

    scale = jnp.where(s == 0, 0.0, 1.0).astype(jnp.float32)
    wblk = wih_ref[...] + scale * whh_ref[...]
    xin = h_ref[...]
    g = lax.dot_general(xin, wblk, (((1,), (1,)), ((), ())),
                        preferred_element_type=jnp.float32)
    gates_ref[pl.ds(b, 1), :] = g + bih_ref[0] + bhh_ref[0]

    @pl.when(b == _NB - 1)
    def _():
        gs = gates_ref[...].reshape(4, FLAT)
        i_ = jax.nn.sigmoid(gs[0:1])
        f_ = jax.nn.sigmoid(gs[1:2])
        g_ = jnp.tanh(gs[2:3])
        o_ = jax.nn.sigmoid(gs[3:4])
        c2 = f_ * c_ref[...] + i_ * g_
        h2 = o_ * jnp.tanh(c2)
        c_ref[...] = c2
        h_ref[...] = h2

        @pl.when(s == 2)
        def _():
            out_ref[...] = h2


def _lstm(w0, W_ih, W_hh, b_ih, b_hh):
    bih2 = b_ih.reshape(_NB, 1, _LB)
    bhh2 = b_hh.reshape(_NB, 1, _LB)
    h = pl.pallas_call(
        _lstm_body,
        grid=(3, _NB),
        in_specs=[
            pl.BlockSpec((1, FLAT), lambda s, b: (0, 0)),
            pl.BlockSpec((_LB, FLAT), lambda s, b: (b, 0)),
            pl.BlockSpec((_LB, FLAT), lambda s, b: (jnp.where(s == 0, 0, b), 0)),
            pl.BlockSpec((1, 1, _LB), lambda s, b: (b, 0, 0)),
            pl.BlockSpec((1, 1, _LB), lambda s, b: (b, 0, 0)),
        ],
        out_specs=pl.BlockSpec((1, FLAT), lambda s, b: (0, 0)),
        out_shape=jax.ShapeDtypeStruct((1, FLAT), jnp.float32),
        scratch_shapes=[
            pltpu.VMEM((_NB, _LB), jnp.float32),
            pltpu.VMEM((1, FLAT), jnp.float32),
            pltpu.VMEM((1, FLAT), jnp.float32),
        ],
    )(w0.reshape(1, FLAT), W_ih, W_hh, bih2, bhh2)
    return h.reshape(D_IN, D_HID)


@functools.partial(
    pl.kernel,
    out_type=jax.ShapeDtypeStruct((32, N_PAD), jnp.float32),
    mesh=_mesh,
    compiler_params=_sc_params,
    scratch_types=[
        pltpu.VMEM((N_PAD,), jnp.float32),
        pltpu.VMEM((DEG_CK,), jnp.int32),
    ],
)
def _deg_kernel(col_hbm, out_hbm, deg_v, idx_v):
    c = lax.axis_index("c")
    s = lax.axis_index("s")
    wid = s * 2 + c
    zeros16 = jnp.zeros((16,), jnp.float32)
    ones16 = jnp.ones((16,), jnp.float32)

    def zb(i, carry):
        deg_v[pl.ds(i * 16, 16)] = zeros16
        return carry

    lax.fori_loop(0, N_PAD // 16, zb, 0)

    base = wid * EPT32

    def chunk(k, carry):
        pltpu.sync_copy(col_hbm.at[pl.ds(base + k * DEG_CK, DEG_CK)], idx_v)

        def inner(j, cc):
            idx16 = idx_v[pl.ds(j * 16, 16)]
            plsc.addupdate_scatter(deg_v, [idx16], ones16)
            return cc

        lax.fori_loop(0, DEG_CK // 16, inner, 0)
        return carry

    lax.fori_loop(0, EPT32 // DEG_CK, chunk, 0)
    pltpu.sync_copy(deg_v, out_hbm.at[wid])


_NBLK = 2944


def _prep_body(x_ref, degp_ref, w_ref, ya_ref, yb_ref, dis_ref):
    deg = jnp.sum(degp_ref[...], axis=0) + 1.0
    dis = lax.rsqrt(deg)
    xw = jnp.dot(x_ref[...], w_ref[...], preferred_element_type=jnp.float32)
    y = dis[:, None] * xw
    ya_ref[...] = y[:, :32]
    yb_ref[...] = y[:, 32:]
    dis_ref[...] = dis[:, None]


def _prep(x_pad, deg_partials, W):
    return pl.pallas_call(
        _prep_body,
        grid=(N_PAD // _NBLK,),
        in_specs=[
            pl.BlockSpec((_NBLK, D_IN), lambda i: (i, 0)),
            pl.BlockSpec((32, _NBLK), lambda i: (0, i)),
            pl.BlockSpec((D_IN, D_HID), lambda i: (0, 0)),
        ],
        out_specs=[
            pl.BlockSpec((_NBLK, 32), lambda i: (i, 0)),
            pl.BlockSpec((_NBLK, 32), lambda i: (i, 0)),
            pl.BlockSpec((_NBLK, 1), lambda i: (i, 0)),
        ],
        out_shape=[
            jax.ShapeDtypeStruct((N_PAD, 32), jnp.float32),
            jax.ShapeDtypeStruct((N_PAD, 32), jnp.float32),
            jax.ShapeDtypeStruct((N_PAD, 1), jnp.float32),
        ],
    )(x_pad, deg_partials, W)


_NCH16 = EPT16 // CK


@functools.partial(
    pl.kernel,
    out_type=(
        jax.ShapeDtypeStruct((N_PAD, 32), jnp.float32),
        jax.ShapeDtypeStruct((N_PAD, 32), jnp.float32),
    ),
    mesh=_mesh,
    compiler_params=_sc_params,
    scratch_types=[
        pltpu.VMEM((2, CK), jnp.int32),
        pltpu.VMEM((2, CK), jnp.int32),
        pltpu.VMEM((2, CK), jnp.int32),
        pltpu.VMEM((2, CK), jnp.int32),
        pltpu.VMEM((CK, 32), jnp.float32),
        pltpu.VMEM((CK, 32), jnp.float32),
        pltpu.VMEM((CK, 32), jnp.float32),
        pltpu.VMEM((CK, 32), jnp.float32),
        pltpu.VMEM_SHARED((N_PAD, 32), jnp.float32),
    ] + [pltpu.SemaphoreType.DMA] * 12,
)
def _agg_kernel(ya_hbm, yb_hbm, ei_hbm, outa_hbm, outb_hbm,
                ib0, ib1, ib2, ib3, gb0, gb1, gb2, gb3, accum,
                sI0, sI1, sI2, sI3, sG0, sG1, sG2, sG3, sS0, sS1, sS2, sS3):
    c = lax.axis_index("c")
    s = lax.axis_index("s")
    ibuf = (ib0, ib1, ib2, ib3)
    semI = (sI0, sI1, sI2, sI3)
    gbuf = (gb0, gb1, gb2, gb3)
    semG = (sG0, sG1, sG2, sG3)
    semS = (sS0, sS1, sS2, sS3)
    zeros16 = jnp.zeros((16,), jnp.float32)

    def zrow(i, carry):
        gb0[i, pl.ds(0, 16)] = zeros16
        gb0[i, pl.ds(16, 16)] = zeros16
        return carry

    lax.fori_loop(0, CK, zrow, 0)

    def zslice(q, carry):
        pltpu.sync_copy(gb0, accum.at[pl.ds(s * NPT + q * CK, CK)])
        return carry

    lax.fori_loop(0, NPT // CK, zslice, 0)
    pltpu.sync_copy(gb0.at[pl.ds(0, NPT % CK)],
                    accum.at[pl.ds(s * NPT + (NPT // CK) * CK, NPT % CK)])
    plsc.subcore_barrier()

    base = s * EPT16

    def start_idx(k, b):
        pltpu.async_copy(ei_hbm.at[:, pl.ds(base + k * CK, CK)], ibuf[b],
                         semI[b])

    def wait_idx(b):
        pltpu.make_async_copy(ei_hbm.at[:, pl.ds(base, CK)], ibuf[b],
                              semI[b]).wait()

    def start_gather(b):
        @pl.when(c == 0)
        def _():
            pltpu.async_copy(ya_hbm.at[ibuf[b].at[0]], gbuf[b], semG[b])

        @pl.when(c == 1)
        def _():
            pltpu.async_copy(yb_hbm.at[ibuf[b].at[0]], gbuf[b], semG[b])

    def wait_gather(b):
        pltpu.make_async_copy(ya_hbm.at[ibuf[b].at[0]], gbuf[b],
                              semG[b]).wait()

    def start_scatter(b):
        pltpu.async_copy(gbuf[b], accum.at[ibuf[b].at[1]], semS[b], add=True)

    def wait_scatter(b):
        pltpu.make_async_copy(gbuf[b], accum.at[ibuf[b].at[1]], semS[b]).wait()

    def steady(k, b, first):
        nb = (b + 1) % 4
        pb = (b - 1) % 4
        wait_gather(b)
        if not first:
            wait_scatter(pb)
        wait_idx(nb)
        start_gather(nb)
        start_scatter(b)
        start_idx(k + 3, pb)

    start_idx(0, 0)
    start_idx(1, 1)
    start_idx(2, 2)
    wait_idx(0)
    start_gather(0)
    steady(0, 0, True)
    steady(1, 1, False)
    steady(2, 2, False)
    steady(3, 3, False)

    def outer(k4, carry):
        k = k4 * 4
        for b in range(4):
            kk = k + b
            nb = (b + 1) % 4
            pb = (b - 1) % 4
            wait_gather(b)
            wait_scatter(pb)
            wait_idx(nb)
            start_gather(nb)
            start_scatter(b)
            pltpu.async_copy(
                ei_hbm.at[:, pl.ds(base + (kk + 3) * CK, CK)], ibuf[pb],
                semI[pb])
        return carry

    lax.fori_loop(1, 97, outer, 0)

    for k in range(388, 392):
        b = k % 4
        nb = (b + 1) % 4
        pb = (b - 1) % 4
        wait_gather(b)
        wait_scatter(pb)
        if k + 1 < _NCH16:
            wait_idx(nb)
            start_gather(nb)
        start_scatter(b)
        if k + 3 < _NCH16:
            start_idx(k + 3, pb)
    wait_scatter(3)

    plsc.subcore_barrier()

    @pl.when(c == 0)
    def _():
        pltpu.sync_copy(accum.at[pl.ds(s * NPT, NPT)],
                        outa_hbm.at[pl.ds(s * NPT, NPT)])

    @pl.when(c == 1)
    def _():
        pltpu.sync_copy(accum.at[pl.ds(s * NPT, NPT)],
                        outb_hbm.at[pl.ds(s * NPT, NPT)])


def _node_mlp_body(agga_ref, aggb_ref, ya_ref, yb_ref, dis_ref,
                   w1s_ref, w1d_ref, b1_ref, asrc_ref, adst_ref):
    agg = jnp.concatenate(
        [agga_ref[...] + ya_ref[...], aggb_ref[...] + yb_ref[...]], axis=1)
    emb = jnp.maximum(dis_ref[...] * agg, 0.0)
    asrc_ref[...] = jnp.dot(emb, w1s_ref[...],
                            preferred_element_type=jnp.float32)
    adst_ref[...] = jnp.dot(emb, w1d_ref[...],
                            preferred_element_type=jnp.float32) + b1_ref[...]


def _node_mlp(agg_a, agg_b, y_a, y_b, dis, W1s, W1d, b1):
    nspec = pl.BlockSpec((_NBLK, 32), lambda i: (i, 0))
    return pl.pallas_call(
        _node_mlp_body,
        grid=(N_PAD // _NBLK,),
        in_specs=[
            nspec, nspec, nspec, nspec,
            pl.BlockSpec((_NBLK, 1), lambda i: (i, 0)),
            pl.BlockSpec((D_HID, D_HID), lambda i: (0, 0)),
            pl.BlockSpec((D_HID, D_HID), lambda i: (0, 0)),
            pl.BlockSpec((1, D_HID), lambda i: (0, 0)),
        ],
        out_specs=[
            pl.BlockSpec((_NBLK, D_HID), lambda i: (i, 0)),
            pl.BlockSpec((_NBLK, D_HID), lambda i: (i, 0)),
        ],
        out_shape=[
            jax.ShapeDtypeStruct((N_PAD, D_HID), jnp.float32),
            jax.ShapeDtypeStruct((N_PAD, D_HID), jnp.float32),
        ],
    )(agg_a, agg_b, y_a, y_b, dis, W1s, W1d, b1.reshape(1, D_HID))


_NCH32 = EPT32 // CK


@functools.partial(
    pl.kernel,
    out_type=jax.ShapeDtypeStruct((E_PAD // 2, 2 * D_HID), jnp.float32),
    mesh=_mesh,
    compiler_params=_sc_params,
    scratch_types=[
        pltpu.VMEM((2, CK), jnp.int32),
        pltpu.VMEM((2, CK), jnp.int32),
        pltpu.VMEM((CK, D_HID), jnp.float32),
        pltpu.VMEM((CK, D_HID), jnp.float32),
        pltpu.VMEM((CK, D_HID), jnp.float32),
        pltpu.VMEM((CK, D_HID), jnp.float32),
        pltpu.VMEM((CK // 2, 2 * D_HID), jnp.float32),
        pltpu.VMEM((CK // 2, 2 * D_HID), jnp.float32),
    ] + [pltpu.SemaphoreType.DMA] * 8,
)
def _edge_gather_kernel(asrc_hbm, adst_hbm, ei_hbm, out_hbm,
                        ib0, ib1, ga0, ga1, gb0, gb1, wb0, wb1,
                        sI0, sI1, sA0, sA1, sB0, sB1, sW0, sW1):
    c = lax.axis_index("c")
    s = lax.axis_index("s")
    wid = s * 2 + c
    base = wid * EPT32
    ibuf = (ib0, ib1)
    gbufa = (ga0, ga1)
    gbufb = (gb0, gb1)
    wbuf = (wb0, wb1)
    semI = (sI0, sI1)
    semA = (sA0, sA1)
    semB = (sB0, sB1)
    semW = (sW0, sW1)

    def start_idx(k, b):
        pltpu.async_copy(ei_hbm.at[:, pl.ds(base + k * CK, CK)], ibuf[b],
                         semI[b])

    def wait_idx(b):
        pltpu.make_async_copy(ei_hbm.at[:, pl.ds(base, CK)], ibuf[b],
                              semI[b]).wait()

    def start_gathers(b):
        pltpu.async_copy(asrc_hbm.at[ibuf[b].at[0]], gbufa[b], semA[b])
        pltpu.async_copy(adst_hbm.at[ibuf[b].at[1]], gbufb[b], semB[b])

    def wait_gathers(b):
        pltpu.make_async_copy(asrc_hbm.at[ibuf[b].at[0]], gbufa[b],
                              semA[b]).wait()
        pltpu.make_async_copy(adst_hbm.at[ibuf[b].at[1]], gbufb[b],
                              semB[b]).wait()

    def start_write(k, b):
        pltpu.async_copy(wbuf[b],
                         out_hbm.at[pl.ds((base + k * CK) // 2, CK // 2)],
                         semW[b])

    def wait_write(b):
        pltpu.make_async_copy(wbuf[b],
                              out_hbm.at[pl.ds(base // 2, CK // 2)],
                              semW[b]).wait()

    def compute(b):
        ga = gbufa[b]
        gb = gbufb[b]
        wb = wbuf[b]

        def add_row(j, cc):
            r = j // 2
            h = (j % 2) * D_HID
            for l in range(D_HID // 16):
                wb[r, pl.ds(h + l * 16, 16)] = (
                    ga[j, pl.ds(l * 16, 16)] + gb[j, pl.ds(l * 16, 16)])
            return cc

        lax.fori_loop(0, CK, add_row, 0)

    def steady(k, b, kfirst):
        o = 1 - b
        wait_gathers(b)
        wait_idx(o)
        start_gathers(o)
        start_idx(k + 2, b)
        if not kfirst:
            wait_write(b)
        compute(b)
        start_write(k, b)

    start_idx(0, 0)
    wait_idx(0)
    start_gathers(0)
    start_idx(1, 1)
    steady(0, 0, True)
    steady(1, 1, True)

    def outer(k2, carry):
        k = k2 * 2
        for b in range(2):
            kk = k + b
            o = 1 - b
            wait_gathers(b)
            wait_idx(o)
            start_gathers(o)
            pltpu.async_copy(ei_hbm.at[:, pl.ds(base + (kk + 2) * CK, CK)],
                             ibuf[b], semI[b])
            wait_write(b)
            compute(b)
            pltpu.async_copy(
                wbuf[b], out_hbm.at[pl.ds((base + kk * CK) // 2, CK // 2)],
                semW[b])
        return carry

    lax.fori_loop(1, 97, outer, 0)

    for k in (194, 195):
        b = k % 2
        o = 1 - b
        wait_gathers(b)
        if k + 1 < _NCH32:
            wait_idx(o)
            start_gathers(o)
        wait_write(b)
        compute(b)
        start_write(k, b)
    wait_write(0)
    wait_write(1)


_EBLK = 6272


def _edge_mlp_body(g_ref, ea_ref, w1e_ref, w2_ref, b2_ref, out_ref):
    eproj = jnp.dot(ea_ref[...], w1e_ref[...],
                    preferred_element_type=jnp.float32)
    hid = jnp.maximum(g_ref[...] + eproj, 0.0)
    out_ref[...] = jnp.dot(hid, w2_ref[...],
                           preferred_element_type=jnp.float32) + b2_ref[...]


def _edge_mlp(g2, eattr2, W1e_stk, W2_stk, b2):
    return pl.pallas_call(
        _edge_mlp_body,
        grid=(E_PAD // 2 // _EBLK,),
        in_specs=[
            pl.BlockSpec((_EBLK, 2 * D_HID), lambda i: (i, 0)),
            pl.BlockSpec((_EBLK, 2 * D_EDGE), lambda i: (i, 0)),
            pl.BlockSpec((2 * D_EDGE, 2 * D_HID), lambda i: (0, 0)),
            pl.BlockSpec((2 * D_HID, 2), lambda i: (0, 0)),
            pl.BlockSpec((1, 1), lambda i: (0, 0)),
        ],
        out_specs=pl.BlockSpec((_EBLK, 2), lambda i: (i, 0)),
        out_shape=jax.ShapeDtypeStruct((E_PAD // 2, 2), jnp.float32),
    )(g2, eattr2, W1e_stk, W2_stk, b2.reshape(1, 1))


def kernel(x_t0, x_t1, x_t2, edge_attr_t0, edge_attr_t1, edge_attr_t2,
           initial_weights, W_ih, W_hh, b_ih, b_hh, W1, b1, W2, b2,
           edge_index_t0, edge_index_t1, edge_index_t2):
    row = edge_index_t2[0]
    col = edge_index_t2[1]
    pad_e = E_PAD - E
    row_pad = jnp.concatenate([row, jnp.zeros((pad_e,), jnp.int32)])
    col_pad = jnp.concatenate([col, jnp.full((pad_e,), N, jnp.int32)])
    ei_pad = jnp.stack([row_pad, col_pad])
    x_pad = jnp.pad(x_t2, ((0, N_PAD - N), (0, 0)))
    eattr2 = jnp.pad(edge_attr_t2, ((0, pad_e), (0, 0))).reshape(
        E_PAD // 2, 2 * D_EDGE)
    W1s = W1[:D_HID]
    W1d = W1[D_HID:2 * D_HID]
    W1e = W1[2 * D_HID:]
    W1e_stk = jnp.zeros((2 * D_EDGE, 2 * D_HID), jnp.float32)
    W1e_stk = W1e_stk.at[:D_EDGE, :D_HID].set(W1e)
    W1e_stk = W1e_stk.at[D_EDGE:, D_HID:].set(W1e)
    W2_stk = jnp.zeros((2 * D_HID, 2), jnp.float32)
    W2_stk = W2_stk.at[:D_HID, 0].set(W2[:, 0])
    W2_stk = W2_stk.at[D_HID:, 1].set(W2[:, 0])

    W = _lstm(initial_weights, W_ih, W_hh, b_ih, b_hh)
    deg_partials = _deg_kernel(col_pad)
    y_a, y_b, dis = _prep(x_pad, deg_partials, W)
    agg_a, agg_b = _agg_kernel(y_a, y_b, ei_pad)
    a_src, a_dst = _node_mlp(agg_a, agg_b, y_a, y_b, dis, W1s, W1d, b1)
    g2 = _edge_gather_kernel(a_src, a_dst, ei_pad)
    logits2 = _edge_mlp(g2, eattr2, W1e_stk, W2_stk, b2)
    return logits2.reshape(E_PAD)[:E]
